# baseline (speedup 1.0000x reference)
import functools

import numpy as np
import jax
import jax.numpy as jnp
from jax import lax
from jax.experimental import pallas as pl
from jax.experimental.pallas import tpu as pltpu
from jax.experimental.pallas import tpu_sc as plsc

_L = 16
_NC, _NS = 2, 16
_NW = _NC * _NS
_RING = 4


def _knn_mean_kernel(Vp, ndim, nfeat, K, C, V, scale):
    assert K % _L == 0 and nfeat % (2 * _L) == 0 and C % _RING == 0
    nh = K // _L
    nj = nfeat // (2 * _L)
    mesh = plsc.VectorSubcoreMesh(
        core_axis_name="c", subcore_axis_name="s",
        num_cores=_NC, num_subcores=_NS)

    @functools.partial(
        pl.kernel,
        mesh=mesh,
        compiler_params=pltpu.CompilerParams(
            needs_layout_passes=False, use_tc_tiling_on_sc=False),
        out_type=jax.ShapeDtypeStruct((Vp * nfeat,), jnp.float32),
        scratch_types=[
            pltpu.VMEM((Vp * ndim,), jnp.float32),
            pltpu.VMEM((C * K,), jnp.int32),
            *([pltpu.VMEM((K, nfeat // 2), jnp.int32)] * _RING),
            *([pltpu.VMEM((nfeat,), jnp.float32)] * _RING),
            pltpu.VMEM_SHARED((V, nfeat // 2), jnp.int32),
            *([pltpu.SemaphoreType.DMA] * (2 * _RING)),
        ],
    )
    def knn(coords_hbm, feat_hbm, idx_hbm, out_hbm,
            cal, idxv, *rest):
        fbs = rest[:_RING]
        obs = rest[_RING:2 * _RING]
        fsh = rest[2 * _RING]
        fsems = rest[2 * _RING + 1:3 * _RING + 1]
        osems = rest[3 * _RING + 1:4 * _RING + 1]
        wid = lax.axis_index("s") * _NC + lax.axis_index("c")
        base = wid * C

        @pl.when(lax.axis_index("s") == 0)
        def _():
            pltpu.sync_copy(feat_hbm, fsh)

        pltpu.sync_copy(coords_hbm, cal)
        pltpu.sync_copy(idx_hbm.at[pl.ds(base * K, C * K)], idxv)
        plsc.subcore_barrier()

        def fire(i, fb, sem):
            pltpu.async_copy(fsh.at[idxv.at[pl.ds(i * K, K)]], fb, sem)

        def wait(fb, sem):
            pltpu.make_async_copy(
                fsh.at[idxv.at[pl.ds(0, K)]], fb, sem).wait()

        def weights(i):
            gi = base + i
            ci = [
                plsc.load_gather(
                    cal, [jnp.full((_L,), gi * ndim + d, jnp.int32)])
                for d in range(ndim)
            ]
            w = []
            for h in range(nh):
                nidx = idxv[pl.ds(i * K + h * _L, _L)]
                na = nidx * ndim
                dist = jnp.zeros((_L,), jnp.float32)
                for d in range(ndim):
                    cn = plsc.load_gather(
                        cal, [na + jnp.full((_L,), d, jnp.int32)])
                    df = cn - ci[d]
                    dist = dist + df * df
                w.append(jnp.exp(dist * (-scale)))
            return w

        def out_start(i, ob, sem):
            pltpu.async_copy(
                ob, out_hbm.at[pl.ds((base + i) * nfeat, nfeat)], sem)

        def out_wait(ob, sem):
            pltpu.make_async_copy(
                ob, out_hbm.at[pl.ds(0, nfeat)], sem).wait()

        def accum(i, w, fb, ob):
            acc_e = [jnp.zeros((_L,), jnp.float32) for _ in range(nj)]
            acc_o = [jnp.zeros((_L,), jnp.float32) for _ in range(nj)]
            for k in range(K):
                wk = jnp.take_along_axis(
                    w[k // _L], jnp.full((_L,), k % _L, jnp.int32), axis=0)
                for j in range(nj):
                    ab32 = fb[k, pl.ds(j * _L, _L)]
                    ab = plsc.bitcast(ab32, jnp.bfloat16)
                    fe, fo = plsc.unpack(
                        ab, format=plsc.PackFormat.INTERLEAVED)
                    acc_e[j] = acc_e[j] + wk * fe
                    acc_o[j] = acc_o[j] + wk * fo
            inv = 1.0 / K
            for j in range(nj):
                ob[pl.ds(j * 2 * _L, _L)] = acc_e[j] * inv
                ob[pl.ds(j * 2 * _L + _L, _L)] = acc_o[j] * inv

        bufs = tuple(zip(fbs, fsems))
        obufs = tuple(zip(obs, osems))
        for a in range(_RING - 1):
            fire(a, *bufs[a])

        def body(t, carry):
            for r in range(_RING):
                i = t * _RING + r
                nfb, nsem = bufs[(r + _RING - 1) % _RING]

                @pl.when(i + _RING - 1 < C)
                def _():
                    fire(i + _RING - 1, nfb, nsem)

                w = weights(i)
                fb, sem = bufs[r]
                wait(fb, sem)
                ob, osem = obufs[r]

                @pl.when(i >= _RING)
                def _():
                    out_wait(ob, osem)

                accum(i, w, fb, ob)
                out_start(i, ob, osem)
            return carry

        lax.fori_loop(0, C // _RING, body, 0)
        for ob, osem in obufs:
            out_wait(ob, osem)

    return knn


def kernel(coordinates, features, neighbour_indices):
    V, ndim = coordinates.shape
    nfeat = features.shape[1]
    K = neighbour_indices.shape[1]
    C = -(-V // (_RING * _NW)) * _RING
    Vp = C * _NW
    coords_pad = (
        jnp.zeros((Vp, ndim), jnp.float32)
        .at[:V].set(coordinates)
        .reshape(Vp * ndim)
    )
    idx_pad = (
        jnp.zeros((Vp, K), jnp.int32)
        .at[:V].set(neighbour_indices)
        .reshape(Vp * K)
    )
    perm = np.arange(nfeat).reshape(-1, 2, _L).transpose(0, 2, 1).reshape(-1)
    feat_bf = features[:, perm].astype(jnp.bfloat16)
    feat_bf = lax.bitcast_convert_type(
        feat_bf.reshape(V, nfeat // 2, 2), jnp.int32)
    knn = _knn_mean_kernel(Vp, ndim, nfeat, K, C, V, scale=10.0)
    f = knn(coords_pad, feat_bf, idx_pad).reshape(Vp, nfeat)[:V]
    return jnp.concatenate([f] * (2 * ndim + 1), axis=1)


# speedup vs baseline: 1.0300x; 1.0300x over previous
import functools

import numpy as np
import jax
import jax.numpy as jnp
from jax import lax
from jax.experimental import pallas as pl
from jax.experimental.pallas import tpu as pltpu
from jax.experimental.pallas import tpu_sc as plsc

_L = 16
_NC, _NS = 2, 16
_NW = _NC * _NS
_RING = 2


def _knn_mean_kernel(Vp, ndim, nfeat, K, C, V, scale):
    assert K % _L == 0 and nfeat % (2 * _L) == 0 and C % _RING == 0
    nh = K // _L
    nj = nfeat // (2 * _L)
    mesh = plsc.VectorSubcoreMesh(
        core_axis_name="c", subcore_axis_name="s",
        num_cores=_NC, num_subcores=_NS)

    @functools.partial(
        pl.kernel,
        mesh=mesh,
        compiler_params=pltpu.CompilerParams(
            needs_layout_passes=False, use_tc_tiling_on_sc=False),
        out_type=jax.ShapeDtypeStruct((Vp * nfeat,), jnp.float32),
        scratch_types=[
            pltpu.VMEM((Vp * ndim,), jnp.float32),
            pltpu.VMEM((C * K,), jnp.int32),
            *([pltpu.VMEM((K, nfeat // 2), jnp.int32)] * _RING),
            *([pltpu.VMEM((nfeat,), jnp.float32)] * _RING),
            pltpu.VMEM_SHARED((V, nfeat // 2), jnp.int32),
            *([pltpu.SemaphoreType.DMA] * (2 * _RING)),
        ],
    )
    def knn(coords_hbm, feat_hbm, idx_hbm, out_hbm,
            cal, idxv, *rest):
        fbs = rest[:_RING]
        obs = rest[_RING:2 * _RING]
        fsh = rest[2 * _RING]
        fsems = rest[2 * _RING + 1:3 * _RING + 1]
        osems = rest[3 * _RING + 1:4 * _RING + 1]
        wid = lax.axis_index("s") * _NC + lax.axis_index("c")
        base = wid * C

        @pl.when(lax.axis_index("s") == 0)
        def _():
            pltpu.sync_copy(feat_hbm, fsh)

        pltpu.sync_copy(coords_hbm, cal)
        pltpu.sync_copy(idx_hbm.at[pl.ds(base * K, C * K)], idxv)
        plsc.subcore_barrier()

        def fire(i, fb, sem):
            pltpu.async_copy(fsh.at[idxv.at[pl.ds(i * K, K)]], fb, sem)

        def wait(fb, sem):
            pltpu.make_async_copy(
                fsh.at[idxv.at[pl.ds(0, K)]], fb, sem).wait()

        def weights(i):
            gi = base + i
            ci = [
                plsc.load_gather(
                    cal, [jnp.full((_L,), gi * ndim + d, jnp.int32)])
                for d in range(ndim)
            ]
            w = []
            for h in range(nh):
                nidx = idxv[pl.ds(i * K + h * _L, _L)]
                na = nidx * ndim
                dist = jnp.zeros((_L,), jnp.float32)
                for d in range(ndim):
                    cn = plsc.load_gather(
                        cal, [na + jnp.full((_L,), d, jnp.int32)])
                    df = cn - ci[d]
                    dist = dist + df * df
                w.append(jnp.exp(dist * (-scale)))
            return w

        def out_start(i, ob, sem):
            pltpu.async_copy(
                ob, out_hbm.at[pl.ds((base + i) * nfeat, nfeat)], sem)

        def out_wait(ob, sem):
            pltpu.make_async_copy(
                ob, out_hbm.at[pl.ds(0, nfeat)], sem).wait()

        def accum(i, w, fb, ob):
            acc_e = [jnp.zeros((_L,), jnp.float32) for _ in range(nj)]
            acc_o = [jnp.zeros((_L,), jnp.float32) for _ in range(nj)]
            for k in range(K):
                wk = jnp.take_along_axis(
                    w[k // _L], jnp.full((_L,), k % _L, jnp.int32), axis=0)
                for j in range(nj):
                    ab32 = fb[k, pl.ds(j * _L, _L)]
                    ab = plsc.bitcast(ab32, jnp.bfloat16)
                    fe, fo = plsc.unpack(
                        ab, format=plsc.PackFormat.INTERLEAVED)
                    acc_e[j] = acc_e[j] + wk * fe
                    acc_o[j] = acc_o[j] + wk * fo
            inv = 1.0 / K
            for j in range(nj):
                ob[pl.ds(j * 2 * _L, _L)] = acc_e[j] * inv
                ob[pl.ds(j * 2 * _L + _L, _L)] = acc_o[j] * inv

        bufs = tuple(zip(fbs, fsems))
        obufs = tuple(zip(obs, osems))
        for a in range(_RING - 1):
            fire(a, *bufs[a])

        def body(t, carry):
            for r in range(_RING):
                i = t * _RING + r
                nfb, nsem = bufs[(r + _RING - 1) % _RING]

                @pl.when(i + _RING - 1 < C)
                def _():
                    fire(i + _RING - 1, nfb, nsem)

                w = weights(i)
                fb, sem = bufs[r]
                wait(fb, sem)
                ob, osem = obufs[r]

                @pl.when(i >= _RING)
                def _():
                    out_wait(ob, osem)

                accum(i, w, fb, ob)
                out_start(i, ob, osem)
            return carry

        lax.fori_loop(0, C // _RING, body, 0)
        for ob, osem in obufs:
            out_wait(ob, osem)

    return knn


def kernel(coordinates, features, neighbour_indices):
    V, ndim = coordinates.shape
    nfeat = features.shape[1]
    K = neighbour_indices.shape[1]
    C = -(-V // (_RING * _NW)) * _RING
    Vp = C * _NW
    coords_pad = (
        jnp.zeros((Vp, ndim), jnp.float32)
        .at[:V].set(coordinates)
        .reshape(Vp * ndim)
    )
    idx_pad = (
        jnp.zeros((Vp, K), jnp.int32)
        .at[:V].set(neighbour_indices)
        .reshape(Vp * K)
    )
    perm = np.arange(nfeat).reshape(-1, 2, _L).transpose(0, 2, 1).reshape(-1)
    feat_bf = features[:, perm].astype(jnp.bfloat16)
    feat_bf = lax.bitcast_convert_type(
        feat_bf.reshape(V, nfeat // 2, 2), jnp.int32)
    knn = _knn_mean_kernel(Vp, ndim, nfeat, K, C, V, scale=10.0)
    f = knn(coords_pad, feat_bf, idx_pad).reshape(Vp, nfeat)[:V]
    return jnp.concatenate([f] * (2 * ndim + 1), axis=1)


# speedup vs baseline: 1.0662x; 1.0352x over previous
import functools

import numpy as np
import jax
import jax.numpy as jnp
from jax import lax
from jax.experimental import pallas as pl
from jax.experimental.pallas import tpu as pltpu
from jax.experimental.pallas import tpu_sc as plsc

_L = 16
_NC, _NS = 2, 16
_NW = _NC * _NS
_RING = 2


def _knn_mean_kernel(Vp, ndim, nfeat, K, C, V, scale):
    assert K % _L == 0 and nfeat % (2 * _L) == 0 and C % _RING == 0
    nh = K // _L
    nj = nfeat // (2 * _L)
    mesh = plsc.VectorSubcoreMesh(
        core_axis_name="c", subcore_axis_name="s",
        num_cores=_NC, num_subcores=_NS)

    @functools.partial(
        pl.kernel,
        mesh=mesh,
        compiler_params=pltpu.CompilerParams(
            needs_layout_passes=False, use_tc_tiling_on_sc=False),
        out_type=jax.ShapeDtypeStruct((Vp * nfeat,), jnp.float32),
        scratch_types=[
            pltpu.VMEM((Vp * ndim,), jnp.float32),
            pltpu.VMEM((C * K,), jnp.int32),
            *([pltpu.VMEM((K, nfeat // 2), jnp.int32)] * _RING),
            *([pltpu.VMEM((nfeat,), jnp.float32)] * _RING),
            pltpu.VMEM_SHARED((V, nfeat // 2), jnp.int32),
            *([pltpu.SemaphoreType.DMA] * (2 * _RING)),
        ],
    )
    def knn(coords_hbm, feat_hbm, idx_hbm, out_hbm,
            cal, idxv, *rest):
        fbs = rest[:_RING]
        obs = rest[_RING:2 * _RING]
        fsh = rest[2 * _RING]
        fsems = rest[2 * _RING + 1:3 * _RING + 1]
        osems = rest[3 * _RING + 1:4 * _RING + 1]
        wid = lax.axis_index("s") * _NC + lax.axis_index("c")
        base = wid * C

        @pl.when(lax.axis_index("s") == 0)
        def _():
            pltpu.sync_copy(feat_hbm, fsh)

        pltpu.sync_copy(coords_hbm, cal)
        pltpu.sync_copy(idx_hbm.at[pl.ds(base * K, C * K)], idxv)
        plsc.subcore_barrier()

        def fire(i, fb, sem):
            pltpu.async_copy(fsh.at[idxv.at[pl.ds(i * K, K)]], fb, sem)

        def wait(fb, sem):
            pltpu.make_async_copy(
                fsh.at[idxv.at[pl.ds(0, K)]], fb, sem).wait()

        def weights(i):
            gi = base + i
            ci = [
                plsc.load_gather(
                    cal, [jnp.full((_L,), gi * ndim + d, jnp.int32)])
                for d in range(ndim)
            ]
            w = []
            for h in range(nh):
                nidx = idxv[pl.ds(i * K + h * _L, _L)]
                na = nidx * ndim
                dist = jnp.zeros((_L,), jnp.float32)
                for d in range(ndim):
                    cn = plsc.load_gather(
                        cal, [na + jnp.full((_L,), d, jnp.int32)])
                    df = cn - ci[d]
                    dist = dist + df * df
                w.append(jnp.exp(dist * (-scale)))
            return w

        def out_start(i, ob, sem):
            pltpu.async_copy(
                ob, out_hbm.at[pl.ds((base + i) * nfeat, nfeat)], sem)

        def out_wait(ob, sem):
            pltpu.make_async_copy(
                ob, out_hbm.at[pl.ds(0, nfeat)], sem).wait()

        def accum(i, w, fb, ob):
            acc_e = [jnp.zeros((_L,), jnp.float32) for _ in range(nj)]
            acc_o = [jnp.zeros((_L,), jnp.float32) for _ in range(nj)]
            for k in range(K):
                wk = jnp.take_along_axis(
                    w[k // _L], jnp.full((_L,), k % _L, jnp.int32), axis=0)
                wkb = plsc.pack(wk, wk, format=plsc.PackFormat.INTERLEAVED)
                for j in range(nj):
                    ab32 = fb[k, pl.ds(j * _L, _L)]
                    ab = plsc.bitcast(ab32, jnp.bfloat16)
                    pe, po = plsc.unpack(
                        ab * wkb, format=plsc.PackFormat.INTERLEAVED)
                    acc_e[j] = acc_e[j] + pe
                    acc_o[j] = acc_o[j] + po
            inv = 1.0 / K
            for j in range(nj):
                ob[pl.ds(j * 2 * _L, _L)] = acc_e[j] * inv
                ob[pl.ds(j * 2 * _L + _L, _L)] = acc_o[j] * inv

        bufs = tuple(zip(fbs, fsems))
        obufs = tuple(zip(obs, osems))
        for a in range(_RING - 1):
            fire(a, *bufs[a])

        def body(t, carry):
            for r in range(_RING):
                i = t * _RING + r
                nfb, nsem = bufs[(r + _RING - 1) % _RING]

                @pl.when(i + _RING - 1 < C)
                def _():
                    fire(i + _RING - 1, nfb, nsem)

                w = weights(i)
                fb, sem = bufs[r]
                wait(fb, sem)
                ob, osem = obufs[r]

                @pl.when(i >= _RING)
                def _():
                    out_wait(ob, osem)

                accum(i, w, fb, ob)
                out_start(i, ob, osem)
            return carry

        lax.fori_loop(0, C // _RING, body, 0)
        for ob, osem in obufs:
            out_wait(ob, osem)

    return knn


def kernel(coordinates, features, neighbour_indices):
    V, ndim = coordinates.shape
    nfeat = features.shape[1]
    K = neighbour_indices.shape[1]
    C = -(-V // (_RING * _NW)) * _RING
    Vp = C * _NW
    coords_pad = (
        jnp.zeros((Vp, ndim), jnp.float32)
        .at[:V].set(coordinates)
        .reshape(Vp * ndim)
    )
    idx_pad = (
        jnp.zeros((Vp, K), jnp.int32)
        .at[:V].set(neighbour_indices)
        .reshape(Vp * K)
    )
    perm = np.arange(nfeat).reshape(-1, 2, _L).transpose(0, 2, 1).reshape(-1)
    feat_bf = features[:, perm].astype(jnp.bfloat16)
    feat_bf = lax.bitcast_convert_type(
        feat_bf.reshape(V, nfeat // 2, 2), jnp.int32)
    knn = _knn_mean_kernel(Vp, ndim, nfeat, K, C, V, scale=10.0)
    f = knn(coords_pad, feat_bf, idx_pad).reshape(Vp, nfeat)[:V]
    return jnp.concatenate([f] * (2 * ndim + 1), axis=1)


# speedup vs baseline: 1.0880x; 1.0204x over previous
import functools

import numpy as np
import jax
import jax.numpy as jnp
from jax import lax
from jax.experimental import pallas as pl
from jax.experimental.pallas import tpu as pltpu
from jax.experimental.pallas import tpu_sc as plsc

_L = 16
_NC, _NS = 2, 16
_NW = _NC * _NS
_RING = 2
_G = 2


def _knn_mean_kernel(Vp, ndim, nfeat, K, C, V, scale):
    assert K % _L == 0 and nfeat % (2 * _L) == 0 and C % (_RING * _G) == 0
    ngrp = C // _G
    nh = K // _L
    nj = nfeat // (2 * _L)
    mesh = plsc.VectorSubcoreMesh(
        core_axis_name="c", subcore_axis_name="s",
        num_cores=_NC, num_subcores=_NS)

    @functools.partial(
        pl.kernel,
        mesh=mesh,
        compiler_params=pltpu.CompilerParams(
            needs_layout_passes=False, use_tc_tiling_on_sc=False),
        out_type=jax.ShapeDtypeStruct((Vp * nfeat,), jnp.float32),
        scratch_types=[
            pltpu.VMEM((Vp * ndim,), jnp.float32),
            pltpu.VMEM((C * K,), jnp.int32),
            *([pltpu.VMEM((_G * K, nfeat // 2), jnp.int32)] * _RING),
            *([pltpu.VMEM((_G * nfeat,), jnp.float32)] * _RING),
            pltpu.VMEM_SHARED((V, nfeat // 2), jnp.int32),
            *([pltpu.SemaphoreType.DMA] * (2 * _RING)),
        ],
    )
    def knn(coords_hbm, feat_hbm, idx_hbm, out_hbm,
            cal, idxv, *rest):
        fbs = rest[:_RING]
        obs = rest[_RING:2 * _RING]
        fsh = rest[2 * _RING]
        fsems = rest[2 * _RING + 1:3 * _RING + 1]
        osems = rest[3 * _RING + 1:4 * _RING + 1]
        wid = lax.axis_index("s") * _NC + lax.axis_index("c")
        base = wid * C

        @pl.when(lax.axis_index("s") == 0)
        def _():
            pltpu.sync_copy(feat_hbm, fsh)

        pltpu.sync_copy(coords_hbm, cal)
        pltpu.sync_copy(idx_hbm.at[pl.ds(base * K, C * K)], idxv)
        plsc.subcore_barrier()

        def fire(g, fb, sem):
            pltpu.async_copy(
                fsh.at[idxv.at[pl.ds(g * _G * K, _G * K)]], fb, sem)

        def wait(fb, sem):
            pltpu.make_async_copy(
                fsh.at[idxv.at[pl.ds(0, _G * K)]], fb, sem).wait()

        def weights(i):
            gi = base + i
            ci = [
                plsc.load_gather(
                    cal, [jnp.full((_L,), gi * ndim + d, jnp.int32)])
                for d in range(ndim)
            ]
            w = []
            for h in range(nh):
                nidx = idxv[pl.ds(i * K + h * _L, _L)]
                na = nidx * ndim
                dist = jnp.zeros((_L,), jnp.float32)
                for d in range(ndim):
                    cn = plsc.load_gather(
                        cal, [na + jnp.full((_L,), d, jnp.int32)])
                    df = cn - ci[d]
                    dist = dist + df * df
                w.append(jnp.exp(dist * (-scale)))
            return w

        def out_start(g, ob, sem):
            pltpu.async_copy(
                ob, out_hbm.at[pl.ds((base + g * _G) * nfeat, _G * nfeat)],
                sem)

        def out_wait(ob, sem):
            pltpu.make_async_copy(
                ob, out_hbm.at[pl.ds(0, _G * nfeat)], sem).wait()

        def accum(v, w, fb, ob):
            acc_e = [jnp.zeros((_L,), jnp.float32) for _ in range(nj)]
            acc_o = [jnp.zeros((_L,), jnp.float32) for _ in range(nj)]
            for k in range(K):
                wk = jnp.take_along_axis(
                    w[k // _L], jnp.full((_L,), k % _L, jnp.int32), axis=0)
                wkb = plsc.pack(wk, wk, format=plsc.PackFormat.INTERLEAVED)
                for j in range(nj):
                    ab32 = fb[v * K + k, pl.ds(j * _L, _L)]
                    ab = plsc.bitcast(ab32, jnp.bfloat16)
                    pe, po = plsc.unpack(
                        ab * wkb, format=plsc.PackFormat.INTERLEAVED)
                    acc_e[j] = acc_e[j] + pe
                    acc_o[j] = acc_o[j] + po
            inv = 1.0 / K
            for j in range(nj):
                ob[pl.ds(v * nfeat + j * 2 * _L, _L)] = acc_e[j] * inv
                ob[pl.ds(v * nfeat + j * 2 * _L + _L, _L)] = acc_o[j] * inv

        bufs = tuple(zip(fbs, fsems))
        obufs = tuple(zip(obs, osems))
        for a in range(_RING - 1):
            fire(a, *bufs[a])

        def body(t, carry):
            for r in range(_RING):
                g = t * _RING + r
                nfb, nsem = bufs[(r + _RING - 1) % _RING]

                @pl.when(g + _RING - 1 < ngrp)
                def _():
                    fire(g + _RING - 1, nfb, nsem)

                ws = [weights(g * _G + v) for v in range(_G)]
                fb, sem = bufs[r]
                wait(fb, sem)
                ob, osem = obufs[r]

                @pl.when(g >= _RING)
                def _():
                    out_wait(ob, osem)

                for v in range(_G):
                    accum(v, ws[v], fb, ob)
                out_start(g, ob, osem)
            return carry

        lax.fori_loop(0, ngrp // _RING, body, 0)
        for ob, osem in obufs:
            out_wait(ob, osem)

    return knn


def kernel(coordinates, features, neighbour_indices):
    V, ndim = coordinates.shape
    nfeat = features.shape[1]
    K = neighbour_indices.shape[1]
    C = -(-V // (_RING * _G * _NW)) * _RING * _G
    Vp = C * _NW
    coords_pad = (
        jnp.zeros((Vp, ndim), jnp.float32)
        .at[:V].set(coordinates)
        .reshape(Vp * ndim)
    )
    idx_pad = (
        jnp.zeros((Vp, K), jnp.int32)
        .at[:V].set(neighbour_indices)
        .reshape(Vp * K)
    )
    perm = np.arange(nfeat).reshape(-1, 2, _L).transpose(0, 2, 1).reshape(-1)
    feat_bf = features[:, perm].astype(jnp.bfloat16)
    feat_bf = lax.bitcast_convert_type(
        feat_bf.reshape(V, nfeat // 2, 2), jnp.int32)
    knn = _knn_mean_kernel(Vp, ndim, nfeat, K, C, V, scale=10.0)
    f = knn(coords_pad, feat_bf, idx_pad).reshape(Vp, nfeat)[:V]
    return jnp.concatenate([f] * (2 * ndim + 1), axis=1)
